# Initial kernel scaffold; baseline (speedup 1.0000x reference)
import jax
import jax.numpy as jnp
from jax.experimental import pallas as pl


def kernel(y_hat, y):
    raise NotImplementedError("write your pallas kernel here")


# speedup vs baseline: 2.1236x; 2.1236x over previous
import functools
import math

import jax
import jax.numpy as jnp
from jax.experimental import pallas as pl
from jax.experimental.pallas import tpu as pltpu

_MARGIN = 0.5
_SCALE = 64.0
_COS_M = math.cos(_MARGIN)
_SIN_M = math.sin(_MARGIN)
_OVERFLOW_THRESH = -math.cos(_MARGIN)
_NEG_HUGE = -1e30


def _margined(t):
    tm = t * _COS_M - jnp.sqrt(jnp.maximum(1.0 - t * t, 0.0)) * _SIN_M
    return jnp.where(t < _OVERFLOW_THRESH, t, tm)


def _body(y_ref, x_ref, out_ref, m_ref, s_ref, t_ref, *, ncb, nclass, cb):
    j = pl.program_id(0)

    @pl.when(j == 0)
    def _init():
        m_ref[...] = jnp.full_like(m_ref, _NEG_HUGE)
        s_ref[...] = jnp.zeros_like(s_ref)
        t_ref[...] = jnp.zeros_like(t_ref)

    x = x_ref[...]
    col = j * cb + jax.lax.broadcasted_iota(jnp.int32, x.shape, 1)
    valid = col < nclass
    is_t = (col == y_ref[...]) & valid

    t_here = jnp.sum(jnp.where(is_t, x, 0.0), axis=1, keepdims=True)
    has_t = jnp.sum(jnp.where(is_t, 1.0, 0.0), axis=1, keepdims=True) > 0
    tm = _margined(t_here) * _SCALE

    z = jnp.where(is_t, tm, x * _SCALE)
    z = jnp.where(valid, z, _NEG_HUGE)

    bm = jnp.max(z, axis=1, keepdims=True)
    m_old = m_ref[...]
    m_new = jnp.maximum(m_old, bm)
    s_ref[...] = s_ref[...] * jnp.exp(m_old - m_new) + jnp.sum(
        jnp.exp(z - m_new), axis=1, keepdims=True)
    m_ref[...] = m_new
    t_ref[...] = t_ref[...] + jnp.where(has_t, tm, 0.0)

    @pl.when(j == ncb - 1)
    def _fin():
        loss_rows = jnp.log(s_ref[...]) + m_ref[...] - t_ref[...]
        out_ref[...] = jnp.sum(loss_rows, axis=(0, 1), keepdims=True) / loss_rows.shape[0]


def _call(y_hat, y, cb, interpret=False):
    b, n = y_hat.shape
    ncb = pl.cdiv(n, cb)
    out = pl.pallas_call(
        functools.partial(_body, ncb=ncb, nclass=n, cb=cb),
        grid=(ncb,),
        in_specs=[
            pl.BlockSpec((b, 1), lambda j: (0, 0)),
            pl.BlockSpec((b, cb), lambda j: (0, j)),
        ],
        out_specs=pl.BlockSpec((1, 1), lambda j: (0, 0)),
        out_shape=jax.ShapeDtypeStruct((1, 1), jnp.float32),
        scratch_shapes=[
            pltpu.VMEM((b, 1), jnp.float32),
            pltpu.VMEM((b, 1), jnp.float32),
            pltpu.VMEM((b, 1), jnp.float32),
        ],
        interpret=interpret,
    )(y.reshape(b, 1), y_hat)
    return out[0, 0]


@jax.jit
def kernel(y_hat, y):
    return _call(y_hat, y, 2048)
